# baseline (speedup 1.0000x reference)
import functools

import jax
import jax.numpy as jnp
import numpy as np
from jax import lax
from jax.experimental import pallas as pl
from jax.experimental.pallas import tpu as pltpu, tpu_sc as plsc

_NC, _NS, _L = 2, 16, 16
_I32_MIN = np.int32(-2147483648)
_I32_MAX = np.int32(2147483647)


def _build(B, S, D):
    seg_len = (B * S) // _NS
    tiles_per_row = _NS // B
    chunks = seg_len // _L
    mesh = plsc.VectorSubcoreMesh(core_axis_name="c", subcore_axis_name="s",
                                  num_cores=_NC, num_subcores=_NS)

    @functools.partial(
        pl.kernel,
        out_type=jax.ShapeDtypeStruct((_L,), jnp.float32),
        mesh=mesh,
        scratch_types=[
            pltpu.VMEM((seg_len,), jnp.int32),
            pltpu.VMEM((_L,), jnp.int32),
            pltpu.VMEM((_NS, _L), jnp.int32),
            pltpu.VMEM((5 * 8,), jnp.float32),
            pltpu.VMEM((_L,), jnp.float32),
            pltpu.VMEM_SHARED((_NS, _L), jnp.int32),
            pltpu.SemaphoreType.DMA,
        ],
        compiler_params=pltpu.CompilerParams(needs_layout_passes=False),
    )
    def sc_kernel(lhs_hbm, ids_hbm, out_hbm, seg_v, pair_v, comb_v, gath_v,
                  outv_v, shared, sem):
        cid = lax.axis_index("c")
        sid = lax.axis_index("s")
        row = sid // tiles_per_row
        seg = sid % tiles_per_row
        seg_base = seg * seg_len
        iota = lax.broadcasted_iota(jnp.int32, (_L,), 0)

        pltpu.sync_copy(ids_hbm.at[pl.ds(row * S + seg_base, seg_len)], seg_v)

        cur_max = jnp.full((_L,), _I32_MIN, jnp.int32)
        cur_idx = jnp.full((_L,), _I32_MAX, jnp.int32)
        for c in range(chunks):
            v = seg_v[pl.ds(c * _L, _L)]
            gidx = iota + (seg_base + c * _L)
            take = v > cur_max
            cur_max = jnp.where(take, v, cur_max)
            cur_idx = jnp.where(take, gidx, cur_idx)
        m = jnp.max(cur_max)
        best = jnp.min(jnp.where(cur_max == m, cur_idx, _I32_MAX))

        pair_v[...] = jnp.where(iota == 0, m, jnp.where(iota == 1, best, 0))
        pltpu.sync_copy(pair_v, shared.at[sid])
        plsc.subcore_barrier()

        @pl.when(sid == 0)
        def _():
            pltpu.sync_copy(shared, comb_v)
            offs = []
            for b in range(B):
                bm = _I32_MIN
                bi = _I32_MAX
                for w in range(b * tiles_per_row, (b + 1) * tiles_per_row):
                    pv = comb_v[w]
                    mv = pv[0]
                    iv = pv[1]
                    better = mv > bm
                    tie = jnp.logical_and(mv == bm, iv < bi)
                    bi = jnp.where(jnp.logical_or(better, tie), iv, bi)
                    bm = jnp.maximum(mv, bm)
                offs.append(b * S * D + bi * (D + 1))
            copies = []
            for b in range(B):
                aligned = (offs[b] // 8) * 8
                copies.append(pltpu.async_copy(
                    lhs_hbm.at[pl.ds(aligned, 8)], gath_v.at[pl.ds(b * 8, 8)],
                    sem))
            for cp in copies:
                cp.wait()
            outv = jnp.zeros((_L,), jnp.float32)
            for b in range(B):
                lane = offs[b] % 8
                win = gath_v[pl.ds(b * 8, _L)]
                val = jnp.sum(jnp.where(iota == lane, win, 0.0))
                outv = jnp.where(iota == b, val * val, outv)
            outv_v[...] = outv

            @pl.when(cid == 0)
            def _():
                pltpu.sync_copy(outv_v, out_hbm)

    return sc_kernel


def kernel(last_hidden_state, input_ids):
    B, S, D = last_hidden_state.shape
    lhs_flat = last_hidden_state.reshape(B * S * D)
    ids_flat = input_ids.astype(jnp.int32).reshape(B * S)
    out16 = _build(B, S, D)(lhs_flat, ids_flat)
    return out16[:B]


# speedup vs baseline: 3.3812x; 3.3812x over previous
import functools

import jax
import jax.numpy as jnp
import numpy as np
from jax import lax
from jax.experimental import pallas as pl
from jax.experimental.pallas import tpu as pltpu, tpu_sc as plsc

_NC, _NS, _L = 2, 16, 16
_I32_MIN = np.int32(-2147483648)
_I32_MAX = np.int32(2147483647)


def _build(B, S, D):
    seg_len = (B * S) // _NS
    tiles_per_row = _NS // B
    chunks = seg_len // _L
    mesh = plsc.VectorSubcoreMesh(core_axis_name="c", subcore_axis_name="s",
                                  num_cores=_NC, num_subcores=_NS)

    @functools.partial(
        pl.kernel,
        out_type=jax.ShapeDtypeStruct((_L,), jnp.float32),
        mesh=mesh,
        scratch_types=[
            pltpu.VMEM((seg_len,), jnp.int32),
            pltpu.VMEM((_L,), jnp.int32),
            pltpu.VMEM((_NS, _L), jnp.int32),
            pltpu.VMEM((B, 128), jnp.float32),
            pltpu.VMEM((_L,), jnp.float32),
            pltpu.VMEM_SHARED((_NS, _L), jnp.int32),
            pltpu.SemaphoreType.DMA,
        ],
        compiler_params=pltpu.CompilerParams(needs_layout_passes=False),
    )
    def sc_kernel(lhs_hbm, ids_hbm, out_hbm, seg_v, pair_v, comb_v, gath_v,
                  outv_v, shared, sem):
        cid = lax.axis_index("c")
        sid = lax.axis_index("s")
        row = sid // tiles_per_row
        seg = sid % tiles_per_row
        seg_base = seg * seg_len
        iota = lax.broadcasted_iota(jnp.int32, (_L,), 0)

        pltpu.sync_copy(ids_hbm.at[row, pl.ds(seg_base, seg_len)], seg_v)

        cur_max = jnp.full((_L,), _I32_MIN, jnp.int32)
        cur_idx = jnp.full((_L,), _I32_MAX, jnp.int32)
        for c in range(chunks):
            v = seg_v[pl.ds(c * _L, _L)]
            gidx = iota + (seg_base + c * _L)
            take = v > cur_max
            cur_max = jnp.where(take, v, cur_max)
            cur_idx = jnp.where(take, gidx, cur_idx)
        m = jnp.max(cur_max)
        best = jnp.min(jnp.where(cur_max == m, cur_idx, _I32_MAX))

        pair_v[...] = jnp.where(iota == 0, m, jnp.where(iota == 1, best, 0))
        pltpu.sync_copy(pair_v, shared.at[sid])
        plsc.subcore_barrier()

        @pl.when(sid == 0)
        def _():
            pltpu.sync_copy(shared, comb_v)
            idxs = []
            for b in range(B):
                bm = _I32_MIN
                bi = _I32_MAX
                for w in range(b * tiles_per_row, (b + 1) * tiles_per_row):
                    pv = comb_v[w]
                    mv = pv[0]
                    iv = pv[1]
                    better = mv > bm
                    tie = jnp.logical_and(mv == bm, iv < bi)
                    bi = jnp.where(jnp.logical_or(better, tie), iv, bi)
                    bm = jnp.maximum(mv, bm)
                idxs.append(bi)
            copies = []
            for b in range(B):
                cs = (idxs[b] // 128) * 128
                copies.append(pltpu.async_copy(
                    lhs_hbm.at[b, idxs[b], pl.ds(cs, 128)], gath_v.at[b], sem))
            for cp in copies:
                cp.wait()
            outv = jnp.zeros((_L,), jnp.float32)
            for b in range(B):
                c16 = ((idxs[b] % 128) // _L) * _L
                lane = idxs[b] % _L
                win = gath_v[b, pl.ds(c16, _L)]
                val = jnp.sum(jnp.where(iota == lane, win, 0.0))
                outv = jnp.where(iota == b, val * val, outv)
            outv_v[...] = outv

            @pl.when(cid == 0)
            def _():
                pltpu.sync_copy(outv_v, out_hbm)

    return sc_kernel


def kernel(last_hidden_state, input_ids):
    B, S, D = last_hidden_state.shape
    out16 = _build(B, S, D)(last_hidden_state, input_ids.astype(jnp.int32))
    return out16[:B]


# speedup vs baseline: 3.6707x; 1.0857x over previous
import functools

import jax
import jax.numpy as jnp
import numpy as np
from jax import lax
from jax.experimental import pallas as pl
from jax.experimental.pallas import tpu as pltpu, tpu_sc as plsc

_NC, _NS, _L = 2, 16, 16
_I32_MIN = np.int32(-2147483648)
_I32_MAX = np.int32(2147483647)


def _build(B, S, D):
    seg_len = (B * S) // _NS
    tiles_per_row = _NS // B
    chunks = seg_len // _L
    mesh = plsc.VectorSubcoreMesh(core_axis_name="c", subcore_axis_name="s",
                                  num_cores=1, num_subcores=_NS)

    @functools.partial(
        pl.kernel,
        out_type=jax.ShapeDtypeStruct((_L,), jnp.float32),
        mesh=mesh,
        scratch_types=[
            pltpu.VMEM((seg_len,), jnp.int32),
            pltpu.VMEM((_L,), jnp.int32),
            pltpu.VMEM((_NS, _L), jnp.int32),
            pltpu.VMEM((B, 128), jnp.float32),
            pltpu.VMEM((_L,), jnp.float32),
            pltpu.VMEM_SHARED((_NS, _L), jnp.int32),
            pltpu.SemaphoreType.DMA,
        ],
        compiler_params=pltpu.CompilerParams(needs_layout_passes=False),
    )
    def sc_kernel(lhs_hbm, ids_hbm, out_hbm, seg_v, pair_v, comb_v, gath_v,
                  outv_v, shared, sem):
        cid = lax.axis_index("c")
        sid = lax.axis_index("s")
        row = sid // tiles_per_row
        seg = sid % tiles_per_row
        seg_base = seg * seg_len
        iota = lax.broadcasted_iota(jnp.int32, (_L,), 0)

        pltpu.sync_copy(ids_hbm.at[row, pl.ds(seg_base, seg_len)], seg_v)

        cur_max = jnp.full((_L,), _I32_MIN, jnp.int32)
        cur_idx = jnp.full((_L,), _I32_MAX, jnp.int32)
        for c in range(chunks):
            v = seg_v[pl.ds(c * _L, _L)]
            gidx = iota + (seg_base + c * _L)
            take = v > cur_max
            cur_max = jnp.where(take, v, cur_max)
            cur_idx = jnp.where(take, gidx, cur_idx)
        m = jnp.max(cur_max)
        best = jnp.min(jnp.where(cur_max == m, cur_idx, _I32_MAX))

        pair_v[...] = jnp.where(iota == 0, m, jnp.where(iota == 1, best, 0))
        pltpu.sync_copy(pair_v, shared.at[sid])
        plsc.subcore_barrier()

        @pl.when(sid == 0)
        def _():
            pltpu.sync_copy(shared, comb_v)
            idxs = []
            for b in range(B):
                bm = _I32_MIN
                bi = _I32_MAX
                for w in range(b * tiles_per_row, (b + 1) * tiles_per_row):
                    pv = comb_v[w]
                    mv = pv[0]
                    iv = pv[1]
                    better = mv > bm
                    tie = jnp.logical_and(mv == bm, iv < bi)
                    bi = jnp.where(jnp.logical_or(better, tie), iv, bi)
                    bm = jnp.maximum(mv, bm)
                idxs.append(bi)
            copies = []
            for b in range(B):
                cs = (idxs[b] // 128) * 128
                copies.append(pltpu.async_copy(
                    lhs_hbm.at[b, idxs[b], pl.ds(cs, 128)], gath_v.at[b], sem))
            for cp in copies:
                cp.wait()
            outv = jnp.zeros((_L,), jnp.float32)
            for b in range(B):
                c16 = ((idxs[b] % 128) // _L) * _L
                lane = idxs[b] % _L
                win = gath_v[b, pl.ds(c16, _L)]
                val = jnp.sum(jnp.where(iota == lane, win, 0.0))
                outv = jnp.where(iota == b, val * val, outv)
            outv_v[...] = outv

            @pl.when(cid == 0)
            def _():
                pltpu.sync_copy(outv_v, out_hbm)

    return sc_kernel


def kernel(last_hidden_state, input_ids):
    B, S, D = last_hidden_state.shape
    out16 = _build(B, S, D)(last_hidden_state, input_ids.astype(jnp.int32))
    return out16[:B]
